# baseline (speedup 1.0000x reference)
import functools

import jax
import jax.numpy as jnp
from jax import lax
from jax.experimental import pallas as pl
from jax.experimental.pallas import tpu as pltpu
from jax.experimental.pallas import tpu_sc as plsc

EMBED = 64
LANES = 16
CHUNK = 128


@functools.lru_cache(maxsize=None)
def _build(B, NC, NS):
  NW = NC * NS
  b_per_w = B // NW
  n_sub = b_per_w // CHUNK
  mesh = plsc.VectorSubcoreMesh(core_axis_name="c", subcore_axis_name="s")

  @functools.partial(
      pl.kernel,
      mesh=mesh,
      compiler_params=pltpu.CompilerParams(use_tc_tiling_on_sc=False),
      out_type=jax.ShapeDtypeStruct((B,), jnp.float32),
      scratch_types=[
          pltpu.VMEM((n_sub, CHUNK), jnp.int32),
          pltpu.VMEM((n_sub, CHUNK), jnp.int32),
          pltpu.VMEM((b_per_w, EMBED), jnp.float32),
          pltpu.VMEM((b_per_w, EMBED), jnp.float32),
          pltpu.VMEM((b_per_w,), jnp.float32),
          pltpu.SemaphoreType.DMA,
      ],
  )
  def k(users_hbm, items_hbm, uemb_hbm, iemb_hbm, out_hbm,
        uidx_v, iidx_v, urows_v, irows_v, out_v, sem):
    wid = lax.axis_index("s") * NC + lax.axis_index("c")
    base = wid * b_per_w
    pltpu.sync_copy(users_hbm.at[wid], uidx_v)
    pltpu.sync_copy(items_hbm.at[wid], iidx_v)

    handles = []
    for j in range(n_sub):
      handles.append(pltpu.async_copy(
          uemb_hbm.at[uidx_v.at[j]],
          urows_v.at[pl.ds(j * CHUNK, CHUNK)], sem))
      handles.append(pltpu.async_copy(
          iemb_hbm.at[iidx_v.at[j]],
          irows_v.at[pl.ds(j * CHUNK, CHUNK)], sem))
    for h in handles:
      h.wait()

    lane = lax.iota(jnp.int32, 16)
    perms = [lane ^ (1 << s) for s in range(4)]
    dnums = lax.GatherDimensionNumbers(
        offset_dims=(), collapsed_slice_dims=(0,), start_index_map=(0,))

    def _shuffle(v, perm):
      return lax.gather(v, perm[:, None], dnums, slice_sizes=(1,),
                        mode=lax.GatherScatterMode.PROMISE_IN_BOUNDS)

    def group(g, carry):
      acc = jnp.zeros((LANES,), jnp.float32)
      for j in range(LANES):
        r = g * LANES + j
        p = urows_v[r, pl.ds(0, 16)] * irows_v[r, pl.ds(0, 16)]
        for c in range(1, EMBED // LANES):
          p = p + urows_v[r, pl.ds(c * 16, 16)] * irows_v[r, pl.ds(c * 16, 16)]
        for s in range(4):
          p = p + _shuffle(p, perms[s])
        acc = jnp.where(lane == j, p, acc)
      out_v[pl.ds(g * LANES, LANES)] = acc
      return carry

    lax.fori_loop(0, b_per_w // LANES, group, 0)
    pltpu.sync_copy(out_v, out_hbm.at[pl.ds(base, b_per_w)])

  return k


def kernel(users, items, user_emb, item_emb_source, item_emb_target):
  del item_emb_target
  B = users.shape[0]
  info = plsc.get_sparse_core_info()
  NC, NS = info.num_cores, info.num_subcores
  NW = NC * NS
  u = users.astype(jnp.int32).reshape(NW, -1, CHUNK)
  it = items.astype(jnp.int32).reshape(NW, -1, CHUNK)
  return _build(B, NC, NS)(u, it, user_emb, item_emb_source)


# speedup vs baseline: 1.0987x; 1.0987x over previous
import functools

import jax
import jax.numpy as jnp
from jax import lax
from jax.experimental import pallas as pl
from jax.experimental.pallas import tpu as pltpu
from jax.experimental.pallas import tpu_sc as plsc

EMBED = 64
FUSED = 2 * EMBED
LANES = 16
CHUNK = 128


@functools.lru_cache(maxsize=None)
def _build(B, NC, NS):
  NW = NC * NS
  b_per_w = B // NW
  n_sub = b_per_w // CHUNK
  mesh = plsc.VectorSubcoreMesh(core_axis_name="c", subcore_axis_name="s")

  @functools.partial(
      pl.kernel,
      mesh=mesh,
      compiler_params=pltpu.CompilerParams(use_tc_tiling_on_sc=False),
      out_type=jax.ShapeDtypeStruct((B,), jnp.float32),
      scratch_types=[
          pltpu.VMEM((n_sub, CHUNK), jnp.int32),
          pltpu.VMEM((n_sub, CHUNK), jnp.int32),
          pltpu.VMEM((CHUNK, FUSED), jnp.float32),
          pltpu.VMEM((CHUNK, FUSED), jnp.float32),
          pltpu.VMEM((b_per_w,), jnp.float32),
          pltpu.SemaphoreType.DMA,
      ],
  )
  def k(users_hbm, items_hbm, fused_hbm, out_hbm,
        uidx_v, iidx_v, urows_v, irows_v, out_v, sem):
    wid = lax.axis_index("s") * NC + lax.axis_index("c")
    base = wid * b_per_w
    pltpu.sync_copy(users_hbm.at[pl.ds(wid * n_sub, n_sub)], uidx_v)
    pltpu.sync_copy(items_hbm.at[pl.ds(wid * n_sub, n_sub)], iidx_v)

    lane = lax.iota(jnp.int32, 16)
    perms = [lane ^ (1 << s) for s in range(4)]
    dnums = lax.GatherDimensionNumbers(
        offset_dims=(), collapsed_slice_dims=(0,), start_index_map=(0,))

    def _shuffle(v, perm):
      return lax.gather(v, perm[:, None], dnums, slice_sizes=(1,),
                        mode=lax.GatherScatterMode.PROMISE_IN_BOUNDS)

    def sub(j, _):
      hu = pltpu.async_copy(fused_hbm.at[uidx_v.at[j]], urows_v, sem)
      hi = pltpu.async_copy(fused_hbm.at[iidx_v.at[j]], irows_v, sem)
      hu.wait()
      hi.wait()

      def group(g, carry):
        acc = jnp.zeros((LANES,), jnp.float32)
        for jj in range(LANES):
          r = g * LANES + jj
          p = (urows_v[r, pl.ds(0, 16)] *
               irows_v[r, pl.ds(EMBED, 16)])
          for c in range(1, EMBED // LANES):
            p = p + (urows_v[r, pl.ds(c * 16, 16)] *
                     irows_v[r, pl.ds(EMBED + c * 16, 16)])
          for s in range(4):
            p = p + _shuffle(p, perms[s])
          acc = jnp.where(lane == jj, p, acc)
        out_v[pl.ds(j * CHUNK + g * LANES, LANES)] = acc
        return carry

      lax.fori_loop(0, CHUNK // LANES, group, 0)
      return _

    lax.fori_loop(0, n_sub, sub, 0)
    pltpu.sync_copy(out_v, out_hbm.at[pl.ds(base, b_per_w)])

  return k


def kernel(users, items, user_emb, item_emb_source, item_emb_target):
  del item_emb_target
  B = users.shape[0]
  info = plsc.get_sparse_core_info()
  NC, NS = info.num_cores, info.num_subcores
  NW = NC * NS
  fused = jnp.concatenate([user_emb, item_emb_source], axis=1)
  u = users.astype(jnp.int32).reshape(NW * (B // NW // CHUNK), CHUNK)
  it = items.astype(jnp.int32).reshape(NW * (B // NW // CHUNK), CHUNK)
  return _build(B, NC, NS)(u, it, fused)
